# Initial kernel scaffold; baseline (speedup 1.0000x reference)
import jax
import jax.numpy as jnp
from jax.experimental import pallas as pl


def kernel(x, edge_index, Wl1, bl1, Wr1, Wl2, bl2, Wr2):
    raise NotImplementedError("write your pallas kernel here")


# speedup vs baseline: 6.2858x; 6.2858x over previous
import functools

import jax
import jax.numpy as jnp
from jax import lax
from jax.experimental import pallas as pl
from jax.experimental.pallas import tpu as pltpu
from jax.experimental.pallas import tpu_sc as plsc

N_NODES = 10000
N_EDGES = 320000
IN_CH = 128
HID_CH = 64
OUT_CH = 2

W1 = 80
W2 = 16

NC = 2
NS = 16
NW = NC * NS
E_PER_W = N_EDGES // NW
K = 80
NCHUNK = E_PER_W // K
RT = 624
ZR = 208
TAIL = N_NODES - NS * RT


def _dot_t(a, b):
    return lax.dot_general(a, b, (((1,), (1,)), ((), ())),
                           preferred_element_type=jnp.float32)


def _tc1_body(x_ref, wl_ref, wr_ref, bl_ref, y1e_ref, z1_ref):
    x = x_ref[...]
    y1 = _dot_t(x, wl_ref[...])
    ones = jnp.ones((N_NODES, 1), jnp.float32)
    pad = jnp.zeros((N_NODES, W1 - HID_CH - 1), jnp.float32)
    y1e_ref[...] = jnp.concatenate([y1, ones, pad], axis=1)
    z1_ref[...] = _dot_t(x, wr_ref[...]) + bl_ref[...]


def _make_seg_sum(width):
    mesh = plsc.VectorSubcoreMesh(core_axis_name="c", subcore_axis_name="s")

    @functools.partial(
        pl.kernel,
        mesh=mesh,
        compiler_params=pltpu.CompilerParams(use_tc_tiling_on_sc=False),
        out_type=jax.ShapeDtypeStruct((NC, N_NODES, width), jnp.float32),
        scratch_types=[
            pltpu.VMEM((K,), jnp.int32),
            pltpu.VMEM((K,), jnp.int32),
            pltpu.VMEM((K, width), jnp.float32),
            pltpu.VMEM((ZR, width), jnp.float32),
            pltpu.VMEM_SHARED((N_NODES, width), jnp.float32),
            pltpu.SemaphoreType.DMA,
        ],
    )
    def seg_sum(table_hbm, src_hbm, dst_hbm, out_hbm,
                idx_s, idx_d, rows, zbuf, acc, sem):
        cid = lax.axis_index("c")
        sid = lax.axis_index("s")
        wid = sid * NC + cid

        zvec = jnp.zeros((16,), jnp.float32)
        cpr = width // 16

        def zero_zbuf(i, carry):
            zbuf[i // cpr, pl.ds((i % cpr) * 16, 16)] = zvec
            return carry
        lax.fori_loop(0, ZR * cpr, zero_zbuf, 0)

        row0 = pl.multiple_of(sid * RT, 8)

        def zero_acc(i, carry):
            off = pl.multiple_of(row0 + i * ZR, 8)
            pltpu.sync_copy(zbuf, acc.at[pl.ds(off, ZR)])
            return carry
        lax.fori_loop(0, RT // ZR, zero_acc, 0)

        @pl.when(sid == NS - 1)
        def _():
            pltpu.sync_copy(zbuf.at[pl.ds(0, TAIL)],
                            acc.at[pl.ds(NS * RT, TAIL)])
        plsc.subcore_barrier()

        base0 = wid * E_PER_W

        def edge_chunk(i, carry):
            base = pl.multiple_of(base0 + i * K, 8)
            pltpu.sync_copy(src_hbm.at[pl.ds(base, K)], idx_s)
            pltpu.sync_copy(dst_hbm.at[pl.ds(base, K)], idx_d)
            pltpu.async_copy(table_hbm.at[idx_s], rows, sem).wait()
            pltpu.sync_copy(rows, acc.at[idx_d], add=True)
            return carry
        lax.fori_loop(0, NCHUNK, edge_chunk, 0)
        plsc.subcore_barrier()

        def readback(i, carry):
            off = pl.multiple_of(row0 + i * ZR, 8)
            pltpu.sync_copy(acc.at[pl.ds(off, ZR)],
                            out_hbm.at[cid].at[pl.ds(off, ZR)])
            return carry
        lax.fori_loop(0, RT // ZR, readback, 0)

        @pl.when(sid == NS - 1)
        def _():
            pltpu.sync_copy(acc.at[pl.ds(NS * RT, TAIL)],
                            out_hbm.at[cid].at[pl.ds(NS * RT, TAIL)])

    return seg_sum


_seg_sum_w1 = _make_seg_sum(W1)
_seg_sum_w2 = _make_seg_sum(W2)


def _tc2_body(p0_ref, p1_ref, z1_ref, wl2_ref, wr2_ref, bl2_ref,
              y2e_ref, z2_ref, deg_ref):
    s = p0_ref[...] + p1_ref[...]
    agg = s[:, :HID_CH]
    deg = jnp.maximum(s[:, HID_CH:HID_CH + 1], 1.0)
    h = jnp.maximum(agg / deg + z1_ref[...], 0.0)
    y2 = _dot_t(h, wl2_ref[...])
    y2e_ref[...] = jnp.concatenate(
        [y2, jnp.zeros((N_NODES, W2 - OUT_CH), jnp.float32)], axis=1)
    z2_ref[...] = _dot_t(h, wr2_ref[...]) + bl2_ref[...]
    deg_ref[...] = deg


def _tc3_body(q0_ref, q1_ref, z2_ref, deg_ref, out_ref):
    s = q0_ref[...] + q1_ref[...]
    o = s[:, :OUT_CH] / deg_ref[...] + z2_ref[...]
    m = jnp.max(o, axis=1, keepdims=True)
    lse = m + jnp.log(jnp.sum(jnp.exp(o - m), axis=1, keepdims=True))
    out_ref[...] = o - lse


def kernel(x, edge_index, Wl1, bl1, Wr1, Wl2, bl2, Wr2):
    src = edge_index[0].astype(jnp.int32)
    dst = edge_index[1].astype(jnp.int32)

    y1e, z1 = pl.pallas_call(
        _tc1_body,
        out_shape=[jax.ShapeDtypeStruct((N_NODES, W1), jnp.float32),
                   jax.ShapeDtypeStruct((N_NODES, HID_CH), jnp.float32)],
    )(x, Wl1, Wr1, bl1.reshape(1, HID_CH))

    acc1 = _seg_sum_w1(y1e, src, dst)

    y2e, z2, deg = pl.pallas_call(
        _tc2_body,
        out_shape=[jax.ShapeDtypeStruct((N_NODES, W2), jnp.float32),
                   jax.ShapeDtypeStruct((N_NODES, OUT_CH), jnp.float32),
                   jax.ShapeDtypeStruct((N_NODES, 1), jnp.float32)],
    )(acc1[0], acc1[1], z1, Wl2, Wr2, bl2.reshape(1, OUT_CH))

    acc2 = _seg_sum_w2(y2e, src, dst)

    out = pl.pallas_call(
        _tc3_body,
        out_shape=jax.ShapeDtypeStruct((N_NODES, OUT_CH), jnp.float32),
    )(acc2[0], acc2[1], z2, deg)
    return out


# speedup vs baseline: 14.6005x; 2.3228x over previous
import functools

import jax
import jax.numpy as jnp
from jax import lax
from jax.experimental import pallas as pl
from jax.experimental.pallas import tpu as pltpu
from jax.experimental.pallas import tpu_sc as plsc

N_NODES = 10000
N_EDGES = 320000
IN_CH = 128
HID_CH = 64
OUT_CH = 2

W1 = 80
W2 = 16

NC = 2
NS = 16
NW = NC * NS
E_PER_W = N_EDGES // NW
K = 104
PIPE = 4
NIT = 24
TAILK = E_PER_W - NIT * PIPE * K
RT = 624
ZR = 208
TAIL = N_NODES - NS * RT


def _dot_t(a, b):
    return lax.dot_general(a, b, (((1,), (1,)), ((), ())),
                           preferred_element_type=jnp.float32)


def _tc1_body(x_ref, wl_ref, wr_ref, bl_ref, y1e_ref, z1_ref):
    x = x_ref[...]
    y1 = _dot_t(x, wl_ref[...])
    ones = jnp.ones((N_NODES, 1), jnp.float32)
    pad = jnp.zeros((N_NODES, W1 - HID_CH - 1), jnp.float32)
    y1e_ref[...] = jnp.concatenate([y1, ones, pad], axis=1)
    z1_ref[...] = _dot_t(x, wr_ref[...]) + bl_ref[...]


def _make_seg_sum(width):
    mesh = plsc.VectorSubcoreMesh(core_axis_name="c", subcore_axis_name="s")

    @functools.partial(
        pl.kernel,
        mesh=mesh,
        compiler_params=pltpu.CompilerParams(use_tc_tiling_on_sc=False),
        out_type=jax.ShapeDtypeStruct((NC, N_NODES, width), jnp.float32),
        scratch_types=[
            [pltpu.VMEM((2, K), jnp.int32) for _ in range(PIPE)],
            [pltpu.VMEM((K, width), jnp.float32) for _ in range(PIPE)],
            pltpu.VMEM((2, TAILK), jnp.int32),
            pltpu.VMEM((ZR, width), jnp.float32),
            pltpu.VMEM_SHARED((N_NODES, width), jnp.float32),
            [pltpu.SemaphoreType.DMA for _ in range(PIPE)],
            [pltpu.SemaphoreType.DMA for _ in range(PIPE)],
            [pltpu.SemaphoreType.DMA for _ in range(PIPE)],
        ],
    )
    def seg_sum(table_hbm, ei_hbm, out_hbm,
                idx, rows, idxt, zbuf, acc, sem_i, sem_g, sem_s):
        cid = lax.axis_index("c")
        sid = lax.axis_index("s")
        wid = sid * NC + cid

        zvec = jnp.zeros((16,), jnp.float32)
        cpr = width // 16

        def zero_zbuf(i, carry):
            zbuf[i // cpr, pl.ds((i % cpr) * 16, 16)] = zvec
            return carry
        lax.fori_loop(0, ZR * cpr, zero_zbuf, 0)

        row0 = pl.multiple_of(sid * RT, 8)

        def zero_acc(i, carry):
            off = pl.multiple_of(row0 + i * ZR, 8)
            pltpu.sync_copy(zbuf, acc.at[pl.ds(off, ZR)])
            return carry
        lax.fori_loop(0, RT // ZR, zero_acc, 0)

        @pl.when(sid == NS - 1)
        def _():
            pltpu.sync_copy(zbuf.at[pl.ds(0, TAIL)],
                            acc.at[pl.ds(NS * RT, TAIL)])
        plsc.subcore_barrier()

        base0 = wid * E_PER_W

        def edge_iter(it, carry):
            cbase = base0 + it * (PIPE * K)
            di = []
            for q in range(PIPE):
                off = pl.multiple_of(cbase + q * K, 8)
                di.append(pltpu.async_copy(
                    ei_hbm.at[:, pl.ds(off, K)], idx[q], sem_i[q]))
            dg = []
            for q in range(PIPE):
                di[q].wait()
                dg.append(pltpu.async_copy(
                    table_hbm.at[idx[q].at[0]], rows[q], sem_g[q]))
            ds = []
            for q in range(PIPE):
                dg[q].wait()
                ds.append(pltpu.async_copy(
                    rows[q], acc.at[idx[q].at[1]], sem_s[q], add=True))
            for q in range(PIPE):
                ds[q].wait()
            return carry
        lax.fori_loop(0, NIT, edge_iter, 0)

        toff = pl.multiple_of(base0 + NIT * PIPE * K, 8)
        pltpu.sync_copy(ei_hbm.at[:, pl.ds(toff, TAILK)], idxt)
        pltpu.async_copy(table_hbm.at[idxt.at[0]],
                         rows[0].at[pl.ds(0, TAILK)], sem_g[0]).wait()
        pltpu.sync_copy(rows[0].at[pl.ds(0, TAILK)],
                        acc.at[idxt.at[1]], add=True)
        plsc.subcore_barrier()

        def readback(i, carry):
            off = pl.multiple_of(row0 + i * ZR, 8)
            pltpu.sync_copy(acc.at[pl.ds(off, ZR)],
                            out_hbm.at[cid].at[pl.ds(off, ZR)])
            return carry
        lax.fori_loop(0, RT // ZR, readback, 0)

        @pl.when(sid == NS - 1)
        def _():
            pltpu.sync_copy(acc.at[pl.ds(NS * RT, TAIL)],
                            out_hbm.at[cid].at[pl.ds(NS * RT, TAIL)])

    return seg_sum


_seg_sum_w1 = _make_seg_sum(W1)
_seg_sum_w2 = _make_seg_sum(W2)


def _tc2_body(p0_ref, p1_ref, z1_ref, wl2_ref, wr2_ref, bl2_ref,
              y2e_ref, z2_ref, deg_ref):
    s = p0_ref[...] + p1_ref[...]
    agg = s[:, :HID_CH]
    deg = jnp.maximum(s[:, HID_CH:HID_CH + 1], 1.0)
    h = jnp.maximum(agg / deg + z1_ref[...], 0.0)
    y2 = _dot_t(h, wl2_ref[...])
    y2e_ref[...] = jnp.concatenate(
        [y2, jnp.zeros((N_NODES, W2 - OUT_CH), jnp.float32)], axis=1)
    z2_ref[...] = _dot_t(h, wr2_ref[...]) + bl2_ref[...]
    deg_ref[...] = deg


def _tc3_body(q0_ref, q1_ref, z2_ref, deg_ref, out_ref):
    s = q0_ref[...] + q1_ref[...]
    o = s[:, :OUT_CH] / deg_ref[...] + z2_ref[...]
    m = jnp.max(o, axis=1, keepdims=True)
    lse = m + jnp.log(jnp.sum(jnp.exp(o - m), axis=1, keepdims=True))
    out_ref[...] = o - lse


def kernel(x, edge_index, Wl1, bl1, Wr1, Wl2, bl2, Wr2):
    ei = edge_index.astype(jnp.int32)

    y1e, z1 = pl.pallas_call(
        _tc1_body,
        out_shape=[jax.ShapeDtypeStruct((N_NODES, W1), jnp.float32),
                   jax.ShapeDtypeStruct((N_NODES, HID_CH), jnp.float32)],
    )(x, Wl1, Wr1, bl1.reshape(1, HID_CH))

    acc1 = _seg_sum_w1(y1e, ei)

    y2e, z2, deg = pl.pallas_call(
        _tc2_body,
        out_shape=[jax.ShapeDtypeStruct((N_NODES, W2), jnp.float32),
                   jax.ShapeDtypeStruct((N_NODES, OUT_CH), jnp.float32),
                   jax.ShapeDtypeStruct((N_NODES, 1), jnp.float32)],
    )(acc1[0], acc1[1], z1, Wl2, Wr2, bl2.reshape(1, OUT_CH))

    acc2 = _seg_sum_w2(y2e, ei)

    out = pl.pallas_call(
        _tc3_body,
        out_shape=jax.ShapeDtypeStruct((N_NODES, OUT_CH), jnp.float32),
    )(acc2[0], acc2[1], z2, deg)
    return out
